# baseline (speedup 1.0000x reference)
import functools

import jax
import jax.numpy as jnp
import numpy as np
from jax import lax
from jax.experimental import pallas as pl
from jax.experimental.pallas import tpu as pltpu
from jax.experimental.pallas import tpu_sc as plsc

N, T = 1048576, 64
NC, NS = 2, 16
NW = NC * NS
ROWS_PER_W = N // NW
CHUNK = 512
NCHUNKS = ROWS_PER_W // CHUNK
NPAIR = NCHUNKS // 2
NG = CHUNK // 16


def _logf(x):
    bits = lax.bitcast_convert_type(x, jnp.int32)
    e = (bits >> 23) - 126
    m = lax.bitcast_convert_type((bits & 0x007FFFFF) | 0x3F000000, jnp.float32)
    c = m < 0.70710678
    e = e - c.astype(jnp.int32)
    t = jnp.where(c, m + m - 1.0, m - 1.0)
    z = t * t
    p = jnp.full((16,), 7.0376836292e-2, jnp.float32)
    for coef in (-1.1514610310e-1, 1.1676998740e-1, -1.2420140846e-1,
                 1.4249322787e-1, -1.6668057665e-1, 2.0000714765e-1,
                 -2.4999993993e-1, 3.3333331174e-1):
        p = p * t + jnp.float32(coef)
    y = p * t * z
    ef = e.astype(jnp.float32)
    y = y + ef * jnp.float32(-2.12194440e-4)
    y = y - 0.5 * z
    return t + y + ef * jnp.float32(0.693359375)


def _sc_body(pflat, labs_h, cen_h, out_h,
             pb0, pb1, lb0, lb1, cb0, cb1, stage, sem0, sem1):
    ci = lax.axis_index("c")
    si = lax.axis_index("s")
    wid = si * NC + ci
    rbase = wid * ROWS_PER_W

    pbs = (pb0, pb1)
    lbs = (lb0, lb1)
    cbs = (cb0, cb1)
    sems = (sem0, sem1)

    def start(c, b):
        off = rbase + c * CHUNK
        pltpu.async_copy(pflat.at[pl.ds(off * T, CHUNK * T)], pbs[b], sems[b])
        pltpu.async_copy(labs_h.at[pl.ds(off, CHUNK)], lbs[b], sems[b])
        pltpu.async_copy(cen_h.at[pl.ds(off, CHUNK)], cbs[b], sems[b])

    def wait(c, b):
        off = rbase + c * CHUNK
        pltpu.make_async_copy(pflat.at[pl.ds(off * T, CHUNK * T)], pbs[b], sems[b]).wait()
        pltpu.make_async_copy(labs_h.at[pl.ds(off, CHUNK)], lbs[b], sems[b]).wait()
        pltpu.make_async_copy(cen_h.at[pl.ds(off, CHUNK)], cbs[b], sems[b]).wait()

    iota64 = lax.iota(jnp.int32, 16) * T

    def process(b, acc):
        pb, lb, cb = pbs[b], lbs[b], cbs[b]

        def gbody(t, acc):
            labs_v = lb[pl.ds(t * 16, 16)]
            cen_v = cb[pl.ds(t * 16, 16)]
            fbase = t * (16 * T) + iota64
            g = plsc.load_gather(pb, [fbase + labs_v]) + jnp.float32(1e-10)
            tail = (63 - labs_v).astype(jnp.float32) * jnp.float32(1e-10)
            for j in range(1, T):
                col = plsc.load_gather(pb, [fbase + j])
                tail = tail + jnp.where(labs_v < j, col, jnp.float32(0.0))
            uncen = cen_v == 0
            val = jnp.where(uncen, g, tail)
            w = jnp.where(uncen, jnp.float32(1.0), jnp.float32(0.5))
            return acc + w * _logf(val)

        return lax.fori_loop(0, NG, gbody, acc)

    start(0, 0)

    def pair(p, acc):
        c0 = 2 * p
        start(c0 + 1, 1)
        wait(c0, 0)
        acc = process(0, acc)

        @pl.when(p < NPAIR - 1)
        def _():
            start(c0 + 2, 0)

        wait(c0 + 1, 1)
        return process(1, acc)

    acc = lax.fori_loop(0, NPAIR, pair, jnp.zeros((16,), jnp.float32))
    stage[...] = acc
    pltpu.sync_copy(stage, out_h.at[pl.ds(wid * 16, 16)])


@jax.jit
def kernel(preds, labs, cen_in):
    mesh = plsc.VectorSubcoreMesh(core_axis_name="c", subcore_axis_name="s")
    partials = pl.kernel(
        _sc_body,
        out_type=jax.ShapeDtypeStruct((NW * 16,), jnp.float32),
        mesh=mesh,
        compiler_params=pltpu.CompilerParams(needs_layout_passes=False),
        scratch_types=[
            pltpu.VMEM((CHUNK * T,), jnp.float32),
            pltpu.VMEM((CHUNK * T,), jnp.float32),
            pltpu.VMEM((CHUNK,), jnp.int32),
            pltpu.VMEM((CHUNK,), jnp.int32),
            pltpu.VMEM((CHUNK,), jnp.int32),
            pltpu.VMEM((CHUNK,), jnp.int32),
            pltpu.VMEM((16,), jnp.float32),
            pltpu.SemaphoreType.DMA,
            pltpu.SemaphoreType.DMA,
        ],
    )(preds.reshape(N * T), labs, cen_in)
    return (-jnp.sum(partials) / np.float32(N)).reshape(1)


# speedup vs baseline: 2.6855x; 2.6855x over previous
import functools

import jax
import jax.numpy as jnp
import numpy as np
from jax import lax
from jax.experimental import pallas as pl
from jax.experimental.pallas import tpu as pltpu
from jax.experimental.pallas import tpu_sc as plsc

N, T = 1048576, 64
NC, NS = 2, 16
NW = NC * NS
CHUNK = 256
R_TC = 786432
R_SC = N - R_TC
ROWS_PER_W = R_SC // NW
NCHUNKS = ROWS_PER_W // CHUNK
NPAIR = NCHUNKS // 2
NG = CHUNK // 16
BR = 8192
assert R_SC % (NW * CHUNK * 2) == 0 and R_TC % BR == 0


def _logf(x):
    bits = lax.bitcast_convert_type(x, jnp.int32)
    e = (bits >> 23) - 126
    m = lax.bitcast_convert_type((bits & 0x007FFFFF) | 0x3F000000, jnp.float32)
    c = m < 0.70710678
    e = e - c.astype(jnp.int32)
    t = jnp.where(c, m + m - 1.0, m - 1.0)
    z = t * t
    p = jnp.full((16,), 7.0376836292e-2, jnp.float32)
    for coef in (-1.1514610310e-1, 1.1676998740e-1, -1.2420140846e-1,
                 1.4249322787e-1, -1.6668057665e-1, 2.0000714765e-1,
                 -2.4999993993e-1, 3.3333331174e-1):
        p = p * t + jnp.float32(coef)
    y = p * t * z
    ef = e.astype(jnp.float32)
    y = y + ef * jnp.float32(-2.12194440e-4)
    y = y - 0.5 * z
    return t + y + ef * jnp.float32(0.693359375)


def _sc_body(preds_h, labs_h, cen_h, out_h,
             pb0, pb1, lb0, lb1, cb0, cb1, stage, ibuf, sem0, sem1):
    ci = lax.axis_index("c")
    si = lax.axis_index("s")
    wid = si * NC + ci
    rbase = R_TC + wid * ROWS_PER_W

    pbs = (pb0, pb1)
    lbs = (lb0, lb1)
    cbs = (cb0, cb1)
    sems = (sem0, sem1)

    def start(c, b):
        off = rbase + c * CHUNK
        pltpu.async_copy(preds_h.at[pl.ds(off, CHUNK)], pbs[b], sems[b])
        pltpu.async_copy(labs_h.at[pl.ds(off, CHUNK)], lbs[b], sems[b])
        pltpu.async_copy(cen_h.at[pl.ds(off, CHUNK)], cbs[b], sems[b])

    def wait(c, b):
        off = rbase + c * CHUNK
        pltpu.make_async_copy(preds_h.at[pl.ds(off, CHUNK)], pbs[b], sems[b]).wait()
        pltpu.make_async_copy(labs_h.at[pl.ds(off, CHUNK)], lbs[b], sems[b]).wait()
        pltpu.make_async_copy(cen_h.at[pl.ds(off, CHUNK)], cbs[b], sems[b]).wait()

    iota = lax.iota(jnp.int32, 16)
    ibuf[...] = iota

    def process(b, acc):
        pb, lb, cb = pbs[b], lbs[b], cbs[b]

        @plsc.parallel_loop(0, NG, carry=acc)
        def gbody(t, acc):
            labs_v = lb[pl.ds(t * 16, 16)]
            cen_v = cb[pl.ds(t * 16, 16)]
            iota_rt = ibuf[...]
            rows = t * 16 + iota_rt
            skew = jnp.bitwise_xor(iota_rt, (iota_rt * 4) & 63)
            g = plsc.load_gather(pb, [rows, labs_v]) + jnp.float32(1e-10)
            zero = jnp.zeros((16,), jnp.float32)
            t0 = (63 - labs_v).astype(jnp.float32) * jnp.float32(1e-10)
            accs = [t0, zero, zero, zero]
            for j in range(T):
                cols = jnp.bitwise_xor(skew, j)
                col = plsc.load_gather(pb, [rows, cols])
                accs[j % 4] = accs[j % 4] + jnp.where(cols > labs_v, col, zero)
            tail = (accs[0] + accs[1]) + (accs[2] + accs[3])
            uncen = cen_v == 0
            val = jnp.where(uncen, g, tail)
            w = jnp.where(uncen, jnp.float32(1.0), jnp.float32(0.5))
            return acc + w * _logf(val)

        return gbody

    start(0, 0)

    def pair(p, acc):
        c0 = 2 * p
        start(c0 + 1, 1)
        wait(c0, 0)
        acc = process(0, acc)

        @pl.when(p < NPAIR - 1)
        def _():
            start(c0 + 2, 0)

        wait(c0 + 1, 1)
        return process(1, acc)

    acc = lax.fori_loop(0, NPAIR, pair, jnp.zeros((16,), jnp.float32))
    stage[...] = acc
    pltpu.sync_copy(stage, out_h.at[pl.ds(wid * 16, 16)])


def _tc_body(enc_ref, preds_ref, out_ref):
    i = pl.program_id(0)
    enc = enc_ref[0]
    labs = enc & 63
    cen = enc >> 6
    pT = jnp.swapaxes(preds_ref[...], 0, 1)
    kio = lax.broadcasted_iota(jnp.int32, (T, BR), 0)
    zero = jnp.zeros((), jnp.float32)
    tail = jnp.sum(jnp.where(kio > labs, pT, zero), axis=0, keepdims=True)
    tail = tail + (63 - labs).astype(jnp.float32) * jnp.float32(1e-10)
    g = jnp.sum(jnp.where(kio == labs, pT, zero), axis=0, keepdims=True)
    g = g + jnp.float32(1e-10)
    uncen = cen == 0
    val = jnp.where(uncen, g, tail)
    w = jnp.where(uncen, jnp.float32(1.0), jnp.float32(0.5))
    contrib = jnp.sum(w * jnp.log(val))

    @pl.when(i == 0)
    def _():
        out_ref[...] = jnp.zeros((1, 1), jnp.float32)

    out_ref[...] += contrib.reshape(1, 1)


def _tc_part(preds, enc):
    enc3 = enc[:R_TC].reshape(R_TC // BR, 1, BR)
    return pl.pallas_call(
        _tc_body,
        grid=(R_TC // BR,),
        in_specs=[
            pl.BlockSpec((1, 1, BR), lambda i: (i, 0, 0)),
            pl.BlockSpec((BR, T), lambda i: (i, 0)),
        ],
        out_specs=pl.BlockSpec((1, 1), lambda i: (0, 0)),
        out_shape=jax.ShapeDtypeStruct((1, 1), jnp.float32),
    )(enc3, preds)


@jax.jit
def kernel(preds, labs, cen_in):
    mesh = plsc.VectorSubcoreMesh(core_axis_name="c", subcore_axis_name="s")
    partials = pl.kernel(
        _sc_body,
        out_type=jax.ShapeDtypeStruct((NW * 16,), jnp.float32),
        mesh=mesh,
        compiler_params=pltpu.CompilerParams(needs_layout_passes=False),
        scratch_types=[
            pltpu.VMEM((CHUNK, T), jnp.float32),
            pltpu.VMEM((CHUNK, T), jnp.float32),
            pltpu.VMEM((CHUNK,), jnp.int32),
            pltpu.VMEM((CHUNK,), jnp.int32),
            pltpu.VMEM((CHUNK,), jnp.int32),
            pltpu.VMEM((CHUNK,), jnp.int32),
            pltpu.VMEM((16,), jnp.float32),
            pltpu.VMEM((16,), jnp.int32),
            pltpu.SemaphoreType.DMA,
            pltpu.SemaphoreType.DMA,
        ],
    )(preds, labs, cen_in)
    enc = labs + T * cen_in
    s_tc = _tc_part(preds, enc)
    total = jnp.sum(partials) + s_tc[0, 0]
    return (-total / np.float32(N)).reshape(1)
